# baseline (speedup 1.0000x reference)
import functools

import jax
import jax.numpy as jnp
from jax import lax
from jax.experimental import pallas as pl
from jax.experimental.pallas import tpu as pltpu
from jax.experimental.pallas import tpu_sc as plsc

N = 10000
D = 256
H = 128
N_PAD = 10240
L = 16
NSC = 2
NTILE = 16
EC = 128
CHUNKS_PER_TILE = 80
E_PAD = EC * CHUNKS_PER_TILE * NTILE
ECHUNKS = E_PAD // EC
STAGES = 2
STAGE_CHUNKS = CHUNKS_PER_TILE // STAGES
DEG_CHUNKS_PER_WORKER = ECHUNKS // (NSC * NTILE)
ROWS_PER_TILE = N_PAD // NTILE
WB = 80

_MESH = plsc.VectorSubcoreMesh(core_axis_name="c", subcore_axis_name="s")


def _deg_body(dst_hbm, deg_out, acc_sh, idx_v, ones_v, stage_v):
    c = lax.axis_index("c")
    s = lax.axis_index("s")
    wid = s * NSC + c
    zeros16 = jnp.zeros((L,), jnp.float32)
    ones16 = jnp.ones((L,), jnp.float32)
    for i in range(128):
        stage_v[i] = zeros16
    for i in range(EC):
        ones_v[i] = ones16
    base = s * ROWS_PER_TILE
    for k in range(ROWS_PER_TILE // 128):
        pltpu.sync_copy(stage_v, acc_sh.at[pl.ds(base + k * 128, 128)])
    plsc.subcore_barrier()
    pltpu.sync_copy(
        dst_hbm.at[pl.ds(wid * DEG_CHUNKS_PER_WORKER, DEG_CHUNKS_PER_WORKER)],
        idx_v)

    def body(j, _):
        pltpu.sync_copy(ones_v, acc_sh.at[idx_v.at[j]], add=True)
        return 0

    lax.fori_loop(0, DEG_CHUNKS_PER_WORKER, body, 0)
    plsc.subcore_barrier()
    for k in range(ROWS_PER_TILE // 128):
        r0 = base + k * 128
        pltpu.sync_copy(acc_sh.at[pl.ds(r0, 128)], stage_v)
        pltpu.sync_copy(stage_v, deg_out.at[pl.ds(c * N_PAD + r0, 128)])


_deg_call = pl.kernel(
    _deg_body,
    out_type=jax.ShapeDtypeStruct((NSC * N_PAD, L), jnp.float32),
    mesh=_MESH,
    scratch_types=[
        pltpu.VMEM_SHARED((N_PAD, L), jnp.float32),
        pltpu.VMEM((DEG_CHUNKS_PER_WORKER, EC), jnp.int32),
        pltpu.VMEM((EC, L), jnp.float32),
        pltpu.VMEM((128, L), jnp.float32),
    ],
)


def _agg_body(hs_hbm, src_hbm, dst_hbm, acc_out,
              acc_sh, src_v, dst_v, rows_a, rows_b,
              sem_a, sem_b):
    c = lax.axis_index("c")
    s = lax.axis_index("s")
    base = s * ROWS_PER_TILE
    cbase = s * CHUNKS_PER_TILE
    rows = (rows_a, rows_b)
    sems = (sem_a, sem_b)
    for layer in range(2):
        slab = 2 * layer + c
        pltpu.sync_copy(
            hs_hbm.at[pl.ds(slab * N_PAD + base, ROWS_PER_TILE)],
            acc_sh.at[pl.ds(base, ROWS_PER_TILE)])
        plsc.subcore_barrier()

        for stage in range(STAGES):
            e0 = cbase + stage * STAGE_CHUNKS
            pltpu.sync_copy(
                src_hbm.at[pl.ds(slab * ECHUNKS + e0, STAGE_CHUNKS)], src_v)
            pltpu.sync_copy(dst_hbm.at[pl.ds(e0, STAGE_CHUNKS)], dst_v)
            for b in range(2):
                pltpu.async_copy(hs_hbm.at[src_v.at[b]], rows[b], sems[b])

            def body(p, _):
                j0 = 2 * p
                for b in range(2):
                    pltpu.make_async_copy(hs_hbm.at[src_v.at[0]],
                                          rows[b], sems[b]).wait()
                    pltpu.sync_copy(rows[b], acc_sh.at[dst_v.at[j0 + b]],
                                    add=True)

                    @pl.when(j0 + b + 2 < STAGE_CHUNKS)
                    def _():
                        pltpu.async_copy(hs_hbm.at[src_v.at[j0 + b + 2]],
                                         rows[b], sems[b])
                return 0

            lax.fori_loop(0, STAGE_CHUNKS // 2, body, 0)
        plsc.subcore_barrier()
        pltpu.sync_copy(
            acc_sh.at[pl.ds(base, ROWS_PER_TILE)],
            acc_out.at[pl.ds(slab * N_PAD + base, ROWS_PER_TILE)])
        plsc.subcore_barrier()


_agg_call = pl.kernel(
    _agg_body,
    out_type=jax.ShapeDtypeStruct((4 * N_PAD, H), jnp.float32),
    mesh=_MESH,
    scratch_types=[
        pltpu.VMEM_SHARED((N_PAD, H), jnp.float32),
        pltpu.VMEM((STAGE_CHUNKS, EC), jnp.int32),
        pltpu.VMEM((STAGE_CHUNKS, EC), jnp.int32),
        pltpu.VMEM((EC, H), jnp.float32),
        pltpu.VMEM((EC, H), jnp.float32),
        pltpu.SemaphoreType.DMA,
        pltpu.SemaphoreType.DMA,
    ],
)

_RMM = 512


def _mm_body(deg_ref, x_ref, wmu_ref, wls_ref, hs_ref):
    deg = deg_ref[0, :, 0] + deg_ref[1, :, 0] + 1.0
    dis = lax.rsqrt(deg)
    xs = x_ref[...] * dis[:, None]
    hmu = jnp.dot(xs, wmu_ref[...], preferred_element_type=jnp.float32)
    hls = jnp.dot(xs, wls_ref[...], preferred_element_type=jnp.float32)
    hs_ref[0] = hmu[:, :H]
    hs_ref[1] = hmu[:, H:]
    hs_ref[2] = hls[:, :H]
    hs_ref[3] = hls[:, H:]


_mm_call = pl.pallas_call(
    _mm_body,
    grid=(N_PAD // _RMM,),
    in_specs=[
        pl.BlockSpec((2, _RMM, L), lambda i: (0, i, 0)),
        pl.BlockSpec((_RMM, D), lambda i: (i, 0)),
        pl.BlockSpec((D, D), lambda i: (0, 0)),
        pl.BlockSpec((D, D), lambda i: (0, 0)),
    ],
    out_specs=pl.BlockSpec((4, _RMM, H), lambda i: (0, i, 0)),
    out_shape=jax.ShapeDtypeStruct((4, N_PAD, H), jnp.float32),
)

_REP = 400


def _ep_body(deg_ref, acc_ref, bmu_ref, bls_ref, omu_ref, ols_ref):
    deg = deg_ref[0, :, 0] + deg_ref[1, :, 0] + 1.0
    dis = lax.rsqrt(deg)[:, None]
    omu_ref[:, :H] = acc_ref[0] * dis + bmu_ref[0, :H]
    omu_ref[:, H:] = acc_ref[1] * dis + bmu_ref[0, H:]
    ols_ref[:, :H] = acc_ref[2] * dis + bls_ref[0, :H]
    ols_ref[:, H:] = acc_ref[3] * dis + bls_ref[0, H:]


_ep_call = pl.pallas_call(
    _ep_body,
    grid=(N // _REP,),
    in_specs=[
        pl.BlockSpec((2, _REP, L), lambda i: (0, i, 0)),
        pl.BlockSpec((4, _REP, H), lambda i: (0, i, 0)),
        pl.BlockSpec((1, D), lambda i: (0, 0)),
        pl.BlockSpec((1, D), lambda i: (0, 0)),
    ],
    out_specs=[
        pl.BlockSpec((_REP, D), lambda i: (i, 0)),
        pl.BlockSpec((_REP, D), lambda i: (i, 0)),
    ],
    out_shape=[
        jax.ShapeDtypeStruct((N, D), jnp.float32),
        jax.ShapeDtypeStruct((N, D), jnp.float32),
    ],
)


@jax.jit
def kernel(x, edge_index, W_mu, b_mu, W_logstd, b_logstd):
    src = edge_index[0]
    dst = edge_index[1]
    npad = E_PAD - src.shape[0]
    pad = jnp.arange(npad, dtype=jnp.int32)
    src_p = jnp.concatenate([src, pad % N])
    src_all = (src_p[None, :]
               + (jnp.arange(4, dtype=jnp.int32) * N_PAD)[:, None]
               ).reshape(4 * ECHUNKS, EC)
    dst_p = jnp.concatenate([dst, N + pad % (N_PAD - N)]).reshape(ECHUNKS, EC)
    x_pad = jnp.pad(x, ((0, N_PAD - N), (0, 0)))

    deg_flat = _deg_call(dst_p)
    deg_st = deg_flat.reshape(NSC, N_PAD, L)
    hs_st = _mm_call(deg_st, x_pad, W_mu, W_logstd)
    acc_flat = _agg_call(hs_st.reshape(4 * N_PAD, H), src_all, dst_p)
    acc_st = acc_flat.reshape(4, N_PAD, H)
    out_mu, out_ls = _ep_call(deg_st, acc_st,
                              b_mu.reshape(1, D), b_logstd.reshape(1, D))
    return out_mu, out_ls
